# baseline (speedup 1.0000x reference)
import functools

import jax
import jax.numpy as jnp
from jax import lax
from jax.experimental import pallas as pl
from jax.experimental.pallas import tpu as pltpu
from jax.experimental.pallas import tpu_sc as plsc

F = 64
LANES = 16
C = 128
NC, NS = 2, 16
NW = NC * NS


def _mf_body(nchunks, nu_half, ufa_hbm, ufb_hbm, itf_hbm, iif_hbm,
             user_hbm, item_hbm, ratings_hbm, logits_hbm,
             idx_u, idx_i, idx_a, idx_b,
             ua0, ub0, it0, iti0, ua1, ub1, it1, iti1, o1, o2,
             sem0, sem1):
    wid = lax.axis_index("s") * NC + lax.axis_index("c")
    per_w = nchunks * C
    wbase = pl.multiple_of(wid * per_w, 8)
    bufs = ((ua0, ub0, it0, iti0, sem0), (ua1, ub1, it1, iti1, sem1))

    pltpu.sync_copy(user_hbm.at[pl.ds(wbase, per_w)], idx_u)
    pltpu.sync_copy(item_hbm.at[pl.ds(wbase, per_w)], idx_i)

    def derive(i, carry):
        s = pl.multiple_of(i * LANES, LANES)
        v = idx_u[pl.ds(s, LANES)]
        idx_a[pl.ds(s, LANES)] = jnp.minimum(v, nu_half - 1)
        idx_b[pl.ds(s, LANES)] = jnp.maximum(v - nu_half, 0)
        return carry

    lax.fori_loop(0, per_w // LANES, derive, 0)

    def fire(g, buf):
        start = pl.multiple_of(g * C, 8)
        uab, ubb, itb, itib, sem = bufs[buf]
        pltpu.async_copy(ufa_hbm.at[idx_a.at[pl.ds(start, C)]], uab, sem)
        pltpu.async_copy(ufb_hbm.at[idx_b.at[pl.ds(start, C)]], ubb, sem)
        pltpu.async_copy(itf_hbm.at[idx_i.at[pl.ds(start, C)]], itb, sem)
        pltpu.async_copy(iif_hbm.at[idx_i.at[pl.ds(start, C)]], itib, sem)

    def drain(buf):
        uab, ubb, itb, itib, sem = bufs[buf]
        pltpu.make_async_copy(ufa_hbm.at[idx_a.at[pl.ds(0, C)]], uab, sem).wait()
        pltpu.make_async_copy(ufb_hbm.at[idx_b.at[pl.ds(0, C)]], ubb, sem).wait()
        pltpu.make_async_copy(itf_hbm.at[idx_i.at[pl.ds(0, C)]], itb, sem).wait()
        pltpu.make_async_copy(iif_hbm.at[idx_i.at[pl.ds(0, C)]], itib, sem).wait()

    lane = lax.iota(jnp.int32, LANES)

    def compute(g, buf):
        uab, ubb, itb, itib, _ = bufs[buf]
        obase = pl.multiple_of(g * C, LANES)

        def group(g2, carry2):
            e0 = pl.multiple_of(g2 * LANES, LANES)
            iu_vec = idx_u[pl.ds(obase + e0, LANES)]
            acc1 = jnp.zeros((LANES,), jnp.float32)
            acc2 = jnp.zeros((LANES,), jnp.float32)
            for j in range(LANES):
                e = e0 + j
                in_a = iu_vec[j] < nu_half
                d1 = jnp.zeros((LANES,), jnp.float32)
                d2 = jnp.zeros((LANES,), jnp.float32)
                for k in range(F // LANES):
                    ua = uab[e, pl.ds(k * LANES, LANES)]
                    ub = ubb[e, pl.ds(k * LANES, LANES)]
                    u = jnp.where(in_a, ua, ub)
                    it = itb[e, pl.ds(k * LANES, LANES)]
                    iti = itib[e, pl.ds(k * LANES, LANES)]
                    d1 = d1 + u * it
                    d2 = d2 + u * iti
                t1 = jnp.sum(d1)
                t2 = jnp.sum(d2)
                mj = lane == j
                acc1 = jnp.where(mj, t1, acc1)
                acc2 = jnp.where(mj, t2, acc2)
            o1[pl.ds(obase + e0, LANES)] = acc1
            o2[pl.ds(obase + e0, LANES)] = acc2
            return carry2

        lax.fori_loop(0, C // LANES, group, 0)

    fire(0, 0)

    def pair(k, carry):
        g = k * 2
        fire(jnp.minimum(g + 1, nchunks - 1), 1)
        drain(0)
        compute(g, 0)
        fire(jnp.minimum(g + 2, nchunks - 1), 0)
        drain(1)
        compute(g + 1, 1)
        return carry

    lax.fori_loop(0, nchunks // 2, pair, 0)
    drain(0)

    pltpu.sync_copy(o1, ratings_hbm.at[pl.ds(wbase, per_w)])
    pltpu.sync_copy(o2, logits_hbm.at[pl.ds(wbase, per_w)])


def kernel(user, item, user_factors, item_factors, item_implicit_factors):
    B, L = user.shape
    BL = B * L
    assert BL % (NW * C) == 0 and (BL // (NW * C)) % 2 == 0
    nchunks = BL // (NW * C)
    per_w = nchunks * C
    nu = user_factors.shape[0]
    assert nu % 2 == 0
    nu_half = nu // 2

    mesh = plsc.VectorSubcoreMesh(core_axis_name="c", subcore_axis_name="s")
    call = pl.kernel(
        functools.partial(_mf_body, nchunks, nu_half),
        out_type=(
            jax.ShapeDtypeStruct((BL,), jnp.float32),
            jax.ShapeDtypeStruct((BL,), jnp.float32),
        ),
        mesh=mesh,
        compiler_params=pltpu.CompilerParams(
            needs_layout_passes=False, use_tc_tiling_on_sc=False
        ),
        scratch_types=[
            pltpu.VMEM((per_w,), jnp.int32),
            pltpu.VMEM((per_w,), jnp.int32),
            pltpu.VMEM((per_w,), jnp.int32),
            pltpu.VMEM((per_w,), jnp.int32),
            pltpu.VMEM((C, F), jnp.float32),
            pltpu.VMEM((C, F), jnp.float32),
            pltpu.VMEM((C, F), jnp.float32),
            pltpu.VMEM((C, F), jnp.float32),
            pltpu.VMEM((C, F), jnp.float32),
            pltpu.VMEM((C, F), jnp.float32),
            pltpu.VMEM((C, F), jnp.float32),
            pltpu.VMEM((C, F), jnp.float32),
            pltpu.VMEM((per_w,), jnp.float32),
            pltpu.VMEM((per_w,), jnp.float32),
            pltpu.SemaphoreType.DMA,
            pltpu.SemaphoreType.DMA,
        ],
    )
    ratings, logits = call(
        user_factors[:nu_half], user_factors[nu_half:],
        item_factors, item_implicit_factors,
        user.reshape(BL), item.reshape(BL),
    )
    return ratings.reshape(B, L), logits.reshape(B, L)


# speedup vs baseline: 7.8999x; 7.8999x over previous
import functools

import jax
import jax.numpy as jnp
from jax import lax
from jax.experimental import pallas as pl
from jax.experimental.pallas import tpu as pltpu
from jax.experimental.pallas import tpu_sc as plsc

F = 64
LANES = 16
C = 128
NC, NS = 2, 16
NW = NC * NS


def _mf_body(nchunks, uf_hbm, itf_hbm, iif_hbm, user_hbm, item_hbm,
             ratings_hbm, logits_hbm,
             idx_u, idx_i, u0, it0, iti0, u1, it1, iti1, o1, o2,
             sem0, sem1):
    wid = lax.axis_index("s") * NC + lax.axis_index("c")
    per_w = nchunks * C
    wbase = pl.multiple_of(wid * per_w, 8)
    bufs = ((u0, it0, iti0, sem0), (u1, it1, iti1, sem1))

    pltpu.sync_copy(user_hbm.at[pl.ds(wbase, per_w)], idx_u)
    pltpu.sync_copy(item_hbm.at[pl.ds(wbase, per_w)], idx_i)

    def fire(g, buf):
        start = pl.multiple_of(g * C, 8)
        ub, itb, itib, sem = bufs[buf]
        pltpu.async_copy(uf_hbm.at[idx_u.at[pl.ds(start, C)]], ub, sem)
        pltpu.async_copy(itf_hbm.at[idx_i.at[pl.ds(start, C)]], itb, sem)
        pltpu.async_copy(iif_hbm.at[idx_i.at[pl.ds(start, C)]], itib, sem)

    def drain(buf):
        ub, itb, itib, sem = bufs[buf]
        pltpu.make_async_copy(uf_hbm.at[idx_u.at[pl.ds(0, C)]], ub, sem).wait()
        pltpu.make_async_copy(itf_hbm.at[idx_i.at[pl.ds(0, C)]], itb, sem).wait()
        pltpu.make_async_copy(iif_hbm.at[idx_i.at[pl.ds(0, C)]], itib, sem).wait()

    lane = lax.iota(jnp.int32, LANES)

    def compute(g, buf):
        ub, itb, itib, _ = bufs[buf]
        obase = pl.multiple_of(g * C, LANES)

        def group(g2, carry2):
            e0 = pl.multiple_of(g2 * LANES, LANES)
            acc1 = jnp.zeros((LANES,), jnp.float32)
            acc2 = jnp.zeros((LANES,), jnp.float32)
            for j in range(LANES):
                e = e0 + j
                d1 = jnp.zeros((LANES,), jnp.float32)
                d2 = jnp.zeros((LANES,), jnp.float32)
                for k in range(F // LANES):
                    u = ub[e, pl.ds(k * LANES, LANES)]
                    it = itb[e, pl.ds(k * LANES, LANES)]
                    iti = itib[e, pl.ds(k * LANES, LANES)]
                    d1 = d1 + u * it
                    d2 = d2 + u * iti
                t1 = jnp.sum(d1)
                t2 = jnp.sum(d2)
                mj = lane == j
                acc1 = jnp.where(mj, t1, acc1)
                acc2 = jnp.where(mj, t2, acc2)
            o1[pl.ds(obase + e0, LANES)] = acc1
            o2[pl.ds(obase + e0, LANES)] = acc2
            return carry2

        lax.fori_loop(0, C // LANES, group, 0)

    fire(0, 0)

    def pair(k, carry):
        g = k * 2
        fire(jnp.minimum(g + 1, nchunks - 1), 1)
        drain(0)
        compute(g, 0)
        fire(jnp.minimum(g + 2, nchunks - 1), 0)
        drain(1)
        compute(g + 1, 1)
        return carry

    lax.fori_loop(0, nchunks // 2, pair, 0)
    drain(0)

    pltpu.sync_copy(o1, ratings_hbm.at[pl.ds(wbase, per_w)])
    pltpu.sync_copy(o2, logits_hbm.at[pl.ds(wbase, per_w)])


def kernel(user, item, user_factors, item_factors, item_implicit_factors):
    B, L = user.shape
    BL = B * L
    assert BL % (NW * C) == 0 and (BL // (NW * C)) % 2 == 0
    nchunks = BL // (NW * C)
    per_w = nchunks * C

    mesh = plsc.VectorSubcoreMesh(core_axis_name="c", subcore_axis_name="s")
    call = pl.kernel(
        functools.partial(_mf_body, nchunks),
        out_type=(
            jax.ShapeDtypeStruct((BL,), jnp.float32),
            jax.ShapeDtypeStruct((BL,), jnp.float32),
        ),
        mesh=mesh,
        compiler_params=pltpu.CompilerParams(
            needs_layout_passes=False, use_tc_tiling_on_sc=False
        ),
        scratch_types=[
            pltpu.VMEM((per_w,), jnp.int32),
            pltpu.VMEM((per_w,), jnp.int32),
            pltpu.VMEM((C, F), jnp.float32),
            pltpu.VMEM((C, F), jnp.float32),
            pltpu.VMEM((C, F), jnp.float32),
            pltpu.VMEM((C, F), jnp.float32),
            pltpu.VMEM((C, F), jnp.float32),
            pltpu.VMEM((C, F), jnp.float32),
            pltpu.VMEM((per_w,), jnp.float32),
            pltpu.VMEM((per_w,), jnp.float32),
            pltpu.SemaphoreType.DMA,
            pltpu.SemaphoreType.DMA,
        ],
    )
    ratings, logits = call(
        user_factors, item_factors, item_implicit_factors,
        user.reshape(BL), item.reshape(BL),
    )
    return ratings.reshape(B, L), logits.reshape(B, L)
